# baseline (speedup 1.0000x reference)
import jax
import jax.numpy as jnp
from jax import lax
from jax.experimental import pallas as pl
from jax.experimental.pallas import tpu as pltpu
from jax.experimental.pallas import tpu_sc as plsc

N = 10000
NPAD = 10240
E = 320000
D = 128
NC = 2
NS = 16
NW = NC * NS
EPT = E // NW
EPAD = 10240
SCK = 128
NPAIR = 20
NSCK = EPAD // 256
ERPW = NSCK * 4
C = 80
EPC = E // NS
RPT = NPAD // NS

_mesh = plsc.VectorSubcoreMesh(core_axis_name="c", subcore_axis_name="s",
                               num_cores=NC, num_subcores=NS)


def _degree_body(edges_hbm, zeros_hbm, deg_hbm, ib, ones_v, acc):
    cid = lax.axis_index("c")
    sid = lax.axis_index("s")
    base = cid * (NW * EPAD // 128) + sid * 160
    row0 = sid * RPT
    pltpu.sync_copy(zeros_hbm.at[pl.ds(row0, RPT)], acc.at[pl.ds(row0, RPT)])
    ones = jnp.full((16,), 1.0, dtype=jnp.float32)

    @pl.loop(0, SCK)
    def _(r):
        for k in range(D // 16):
            ones_v[r, pl.ds(k * 16, 16)] = ones

    plsc.subcore_barrier()

    @pl.loop(0, 20)
    def _(p):
        pltpu.sync_copy(edges_hbm.at[pl.ds(base + p * 8, 8)], ib)
        for u in range(8):
            pltpu.sync_copy(ones_v, acc.at[ib.at[u]], add=True)

    plsc.subcore_barrier()
    pltpu.sync_copy(acc.at[pl.ds(row0, RPT)], deg_hbm.at[cid, pl.ds(row0, RPT)])


_deg_call = pl.kernel(
    _degree_body,
    out_type=jax.ShapeDtypeStruct((NC, NPAD, D), jnp.float32),
    mesh=_mesh,
    scratch_types=[
        pltpu.VMEM((8, 128), jnp.int32),
        pltpu.VMEM((SCK, D), jnp.float32),
        pltpu.VMEM_SHARED((NPAD, D), jnp.float32),
    ],
)


def _agg_body(x_hbm, edges_hbm, zeros_hbm, out_hbm, ib, rb, acc):
    cid = lax.axis_index("c")
    sid = lax.axis_index("s")
    wid = cid * NS + sid
    ebase = wid * ERPW
    row0 = sid * RPT
    pltpu.sync_copy(zeros_hbm.at[pl.ds(row0, RPT)], acc.at[pl.ds(row0, RPT)])
    plsc.subcore_barrier()

    GROW = (0, 1, 4, 5)
    SROW = (2, 3, 6, 7)

    @pl.loop(0, NPAIR)
    def _(p):
        pltpu.sync_copy(edges_hbm.at[pl.ds(ebase + p * 8, 8)], ib)
        for u in range(4):
            pltpu.sync_copy(x_hbm.at[ib.at[GROW[u]]], rb)
            pltpu.sync_copy(rb, acc.at[ib.at[SROW[u]]], add=True)

    plsc.subcore_barrier()
    pltpu.sync_copy(acc.at[pl.ds(row0, RPT)], out_hbm.at[cid, pl.ds(row0, RPT)])


_agg_call = pl.kernel(
    _agg_body,
    out_type=jax.ShapeDtypeStruct((NC, NPAD, D), jnp.float32),
    mesh=_mesh,
    scratch_types=[
        pltpu.VMEM((8, 128), jnp.int32),
        pltpu.VMEM((SCK, D), jnp.float32),
        pltpu.VMEM_SHARED((NPAD, D), jnp.float32),
    ],
)


def _prep_body(feat_ref, deg_ref, xs_ref, ns_ref, nd_ref):
    d_o = deg_ref[0, :N, 0:1]
    d_i = deg_ref[1, :N, 0:1]
    ns = jnp.where(d_o > 0, lax.rsqrt(d_o), 0.0)
    nd = jnp.where(d_i > 0, lax.rsqrt(d_i), 0.0)
    ns_ref[...] = ns
    nd_ref[...] = nd
    xs_ref[0:N, :] = feat_ref[...] * ns
    xs_ref[N:NPAD, :] = jnp.zeros((NPAD - N, D), jnp.float32)


_prep_call = pl.pallas_call(
    _prep_body,
    out_shape=(jax.ShapeDtypeStruct((NPAD, D), jnp.float32),
               jax.ShapeDtypeStruct((N, 1), jnp.float32),
               jax.ShapeDtypeStruct((N, 1), jnp.float32)),
)


def _layer1_body(p_ref, nd_ref, ns_ref, w_ref, b_ref, o_ref):
    t = (p_ref[0, :N] + p_ref[1, :N]) * nd_ref[...]
    h = jnp.dot(t, w_ref[...], preferred_element_type=jnp.float32) + b_ref[...]
    o_ref[0:N, :] = jnp.maximum(h, 0.0) * ns_ref[...]
    o_ref[N:NPAD, :] = jnp.zeros((NPAD - N, D), jnp.float32)


_layer1_call = pl.pallas_call(
    _layer1_body,
    out_shape=jax.ShapeDtypeStruct((NPAD, D), jnp.float32),
)


def _layer2_body(p_ref, nd_ref, w_ref, b_ref, o_ref):
    t = (p_ref[0, :N] + p_ref[1, :N]) * nd_ref[...]
    o_ref[...] = jnp.dot(t, w_ref[...], preferred_element_type=jnp.float32) + b_ref[...]


_layer2_call = pl.pallas_call(
    _layer2_body,
    out_shape=jax.ShapeDtypeStruct((N, D), jnp.float32),
)


def _pack_deg_rows(src, dst):
    pad = ((0, 0), (0, EPAD - EPT))
    s = jnp.pad(src.reshape(NW, EPT), pad, constant_values=NPAD - 1)
    d = jnp.pad(dst.reshape(NW, EPT), pad, constant_values=NPAD - 1)
    return jnp.concatenate([s.reshape(-1, 128), d.reshape(-1, 128)])


def _pack_edges(src, dst):
    pad = ((0, 0), (0, EPAD - EPT))
    s = jnp.pad(src.reshape(NW, EPT), pad, constant_values=NPAD - 1)
    d = jnp.pad(dst.reshape(NW, EPT), pad, constant_values=NPAD - 1)
    s = s.reshape(NW, NSCK, 2, 128)
    d = d.reshape(NW, NSCK, 2, 128)
    return jnp.concatenate([s, d], axis=2).reshape(NW * NSCK * 4, 128)


def kernel(feat, edge_index, W1, b1, W2, b2):
    src = edge_index[0].astype(jnp.int32)
    dst = edge_index[1].astype(jnp.int32)
    edges_packed = _pack_edges(src, dst)
    zeros_nd = jnp.zeros((NPAD, D), jnp.float32)

    deg = _deg_call(_pack_deg_rows(src, dst), zeros_nd)
    xs, ns, nd = _prep_call(feat, deg)
    p1 = _agg_call(xs, edges_packed, zeros_nd)
    x2 = _layer1_call(p1, nd, ns, W1, b1.reshape(1, D))
    p2 = _agg_call(x2, edges_packed, zeros_nd)
    return _layer2_call(p2, nd, W2, b2.reshape(1, D))


# speedup vs baseline: 1.7618x; 1.7618x over previous
import jax
import jax.numpy as jnp
from jax import lax
from jax.experimental import pallas as pl
from jax.experimental.pallas import tpu as pltpu
from jax.experimental.pallas import tpu_sc as plsc

N = 10000
NPAD = 10240
E = 320000
D = 128
NC = 2
NS = 16
NW = NC * NS
EPT = E // NW
C = 80
EPC = E // NS
RPT = NPAD // NS
NCHUNK = EPT // C

_mesh = plsc.VectorSubcoreMesh(core_axis_name="c", subcore_axis_name="s",
                               num_cores=NC, num_subcores=NS)


def _degree_body(edges_hbm, zeros_hbm, deg_hbm,
                 sidx, ones_v, acc):
    cid = lax.axis_index("c")
    sid = lax.axis_index("s")
    base = cid * E + sid * EPC
    row0 = sid * RPT
    pltpu.sync_copy(zeros_hbm.at[pl.ds(row0, RPT)], acc.at[pl.ds(row0, RPT)])
    ones = jnp.full((16,), 1.0, dtype=jnp.float32)

    @pl.loop(0, C)
    def _(r):
        for k in range(D // 16):
            ones_v[r, pl.ds(k * 16, 16)] = ones

    plsc.subcore_barrier()

    @pl.loop(0, EPC, step=C)
    def _(i):
        pltpu.sync_copy(edges_hbm.at[pl.ds(base + i, C)], sidx)
        pltpu.sync_copy(ones_v, acc.at[sidx], add=True)

    plsc.subcore_barrier()
    pltpu.sync_copy(acc.at[pl.ds(row0, RPT)], deg_hbm.at[cid, pl.ds(row0, RPT)])


_deg_call = pl.kernel(
    _degree_body,
    out_type=jax.ShapeDtypeStruct((NC, NPAD, D), jnp.float32),
    mesh=_mesh,
    scratch_types=[
        pltpu.VMEM((C,), jnp.int32),
        pltpu.VMEM((C, D), jnp.float32),
        pltpu.VMEM_SHARED((NPAD, D), jnp.float32),
    ],
)


def _agg_body(x_hbm, src_hbm, dst_hbm, zeros_hbm, out_hbm,
              sidx0, didx0, sidx1, didx1, rows0, rows1, acc, gs0, gs1):
    cid = lax.axis_index("c")
    sid = lax.axis_index("s")
    base = (cid * NS + sid) * EPT
    row0 = sid * RPT
    pltpu.sync_copy(zeros_hbm.at[pl.ds(row0, RPT)], acc.at[pl.ds(row0, RPT)])

    sxs = (sidx0, sidx1)
    dxs = (didx0, didx1)
    rbs = (rows0, rows1)
    gss = (gs0, gs1)

    def idx_sync(i, j):
        pltpu.sync_copy(src_hbm.at[pl.ds(base + i * C, C)], sxs[j])
        pltpu.sync_copy(dst_hbm.at[pl.ds(base + i * C, C)], dxs[j])

    def gather_issue(j):
        pltpu.async_copy(x_hbm.at[sxs[j]], rbs[j], gss[j])

    def gather_wait(j):
        pltpu.make_async_copy(x_hbm.at[pl.ds(0, C)], rbs[j], gss[j]).wait()

    def scatter(j):
        pltpu.sync_copy(rbs[j], acc.at[dxs[j]], add=True)

    idx_sync(0, 0)
    gather_issue(0)
    plsc.subcore_barrier()

    @pl.loop(0, NCHUNK - 1, step=2)
    def _(a):
        idx_sync(a + 1, 1)
        gather_issue(1)
        gather_wait(0)
        scatter(0)
        idx_sync(a + 2, 0)
        gather_issue(0)
        gather_wait(1)
        scatter(1)

    gather_wait(0)
    scatter(0)
    plsc.subcore_barrier()
    pltpu.sync_copy(acc.at[pl.ds(row0, RPT)], out_hbm.at[cid, pl.ds(row0, RPT)])


_agg_call = pl.kernel(
    _agg_body,
    out_type=jax.ShapeDtypeStruct((NC, NPAD, D), jnp.float32),
    mesh=_mesh,
    scratch_types=[
        pltpu.VMEM((C,), jnp.int32),
        pltpu.VMEM((C,), jnp.int32),
        pltpu.VMEM((C,), jnp.int32),
        pltpu.VMEM((C,), jnp.int32),
        pltpu.VMEM((C, D), jnp.float32),
        pltpu.VMEM((C, D), jnp.float32),
        pltpu.VMEM_SHARED((NPAD, D), jnp.float32),
        pltpu.SemaphoreType.DMA,
        pltpu.SemaphoreType.DMA,
    ],
)


def _prep_body(feat_ref, deg_ref, xs_ref, ns_ref, nd_ref):
    d_o = deg_ref[0, :N, 0:1]
    d_i = deg_ref[1, :N, 0:1]
    ns = jnp.where(d_o > 0, lax.rsqrt(d_o), 0.0)
    nd = jnp.where(d_i > 0, lax.rsqrt(d_i), 0.0)
    ns_ref[...] = ns
    nd_ref[...] = nd
    xs_ref[...] = feat_ref[...] * ns


_prep_call = pl.pallas_call(
    _prep_body,
    out_shape=(jax.ShapeDtypeStruct((N, D), jnp.float32),
               jax.ShapeDtypeStruct((N, 1), jnp.float32),
               jax.ShapeDtypeStruct((N, 1), jnp.float32)),
)


def _layer1_body(p_ref, nd_ref, ns_ref, w_ref, b_ref, o_ref):
    t = (p_ref[0, :N] + p_ref[1, :N]) * nd_ref[...]
    h = jnp.dot(t, w_ref[...], preferred_element_type=jnp.float32) + b_ref[...]
    o_ref[...] = jnp.maximum(h, 0.0) * ns_ref[...]


_layer1_call = pl.pallas_call(
    _layer1_body,
    out_shape=jax.ShapeDtypeStruct((N, D), jnp.float32),
)


def _layer2_body(p_ref, nd_ref, w_ref, b_ref, o_ref):
    t = (p_ref[0, :N] + p_ref[1, :N]) * nd_ref[...]
    o_ref[...] = jnp.dot(t, w_ref[...], preferred_element_type=jnp.float32) + b_ref[...]


_layer2_call = pl.pallas_call(
    _layer2_body,
    out_shape=jax.ShapeDtypeStruct((N, D), jnp.float32),
)


def kernel(feat, edge_index, W1, b1, W2, b2):
    src = edge_index[0].astype(jnp.int32)
    dst = edge_index[1].astype(jnp.int32)
    zeros_nd = jnp.zeros((NPAD, D), jnp.float32)

    deg = _deg_call(edge_index.reshape(-1).astype(jnp.int32), zeros_nd)
    xs, ns, nd = _prep_call(feat, deg)
    p1 = _agg_call(xs, src, dst, zeros_nd)
    x2 = _layer1_call(p1, nd, ns, W1, b1.reshape(1, D))
    p2 = _agg_call(x2, src, dst, zeros_nd)
    return _layer2_call(p2, nd, W2, b2.reshape(1, D))


# speedup vs baseline: 2.0911x; 1.1869x over previous
import jax
import jax.numpy as jnp
from jax import lax
from jax.experimental import pallas as pl
from jax.experimental.pallas import tpu as pltpu
from jax.experimental.pallas import tpu_sc as plsc

N = 10000
NPAD = 10240
E = 320000
D = 128
NC = 2
NS = 16
NW = NC * NS
EPT = E // NW
C = 80
EPC = E // NS
RPT = NPAD // NS
NCHUNK = EPT // C

_mesh = plsc.VectorSubcoreMesh(core_axis_name="c", subcore_axis_name="s",
                               num_cores=NC, num_subcores=NS)


def _degree_body(edges_hbm, zeros_hbm, deg_hbm,
                 sidx0, sidx1, ones_v, acc, is0, is1):
    cid = lax.axis_index("c")
    sid = lax.axis_index("s")
    base = cid * E + sid * EPC
    row0 = sid * RPT
    pltpu.sync_copy(zeros_hbm.at[pl.ds(row0, RPT)], acc.at[pl.ds(row0, RPT)])
    ones = jnp.full((16,), 1.0, dtype=jnp.float32)

    @pl.loop(0, C)
    def _(r):
        for k in range(D // 16):
            ones_v[r, pl.ds(k * 16, 16)] = ones

    sxs = (sidx0, sidx1)
    iss = (is0, is1)
    NDCH = EPC // C

    def idx_issue(i, j):
        pltpu.async_copy(edges_hbm.at[pl.ds(base + i * C, C)], sxs[j], iss[j])

    def idx_wait(j):
        pltpu.make_async_copy(edges_hbm.at[pl.ds(0, C)], sxs[j], iss[j]).wait()

    idx_issue(0, 0)
    plsc.subcore_barrier()

    @pl.loop(0, NDCH, step=2)
    def _(a):
        idx_issue(a + 1, 1)
        idx_wait(0)
        pltpu.sync_copy(ones_v, acc.at[sidx0], add=True)
        idx_issue(jnp.minimum(a + 2, NDCH - 1), 0)
        idx_wait(1)
        pltpu.sync_copy(ones_v, acc.at[sidx1], add=True)

    idx_wait(0)
    plsc.subcore_barrier()
    pltpu.sync_copy(acc.at[pl.ds(row0, RPT)], deg_hbm.at[cid, pl.ds(row0, RPT)])


_deg_call = pl.kernel(
    _degree_body,
    out_type=jax.ShapeDtypeStruct((NC, NPAD, D), jnp.float32),
    mesh=_mesh,
    scratch_types=[
        pltpu.VMEM((C,), jnp.int32),
        pltpu.VMEM((C,), jnp.int32),
        pltpu.VMEM((C, D), jnp.float32),
        pltpu.VMEM_SHARED((NPAD, D), jnp.float32),
        pltpu.SemaphoreType.DMA,
        pltpu.SemaphoreType.DMA,
    ],
)


def _agg_body(x_hbm, src_hbm, dst_hbm, zeros_hbm, out_hbm,
              sidx0, didx0, sidx1, didx1, rows0, rows1, acc, gs0, gs1):
    cid = lax.axis_index("c")
    sid = lax.axis_index("s")
    base = (cid * NS + sid) * EPT
    row0 = sid * RPT
    pltpu.sync_copy(zeros_hbm.at[pl.ds(row0, RPT)], acc.at[pl.ds(row0, RPT)])

    sxs = (sidx0, sidx1)
    dxs = (didx0, didx1)
    rbs = (rows0, rows1)
    gss = (gs0, gs1)

    def idx_sync(i, j):
        pltpu.sync_copy(src_hbm.at[pl.ds(base + i * C, C)], sxs[j])
        pltpu.sync_copy(dst_hbm.at[pl.ds(base + i * C, C)], dxs[j])

    def gather_issue(j):
        pltpu.async_copy(x_hbm.at[sxs[j]], rbs[j], gss[j])

    def gather_wait(j):
        pltpu.make_async_copy(x_hbm.at[pl.ds(0, C)], rbs[j], gss[j]).wait()

    def scatter(j):
        pltpu.sync_copy(rbs[j], acc.at[dxs[j]], add=True)

    idx_sync(0, 0)
    gather_issue(0)
    plsc.subcore_barrier()

    @pl.loop(0, NCHUNK - 1, step=2)
    def _(a):
        idx_sync(a + 1, 1)
        gather_issue(1)
        gather_wait(0)
        scatter(0)
        idx_sync(a + 2, 0)
        gather_issue(0)
        gather_wait(1)
        scatter(1)

    gather_wait(0)
    scatter(0)
    plsc.subcore_barrier()
    pltpu.sync_copy(acc.at[pl.ds(row0, RPT)], out_hbm.at[cid, pl.ds(row0, RPT)])


_agg_call = pl.kernel(
    _agg_body,
    out_type=jax.ShapeDtypeStruct((NC, NPAD, D), jnp.float32),
    mesh=_mesh,
    scratch_types=[
        pltpu.VMEM((C,), jnp.int32),
        pltpu.VMEM((C,), jnp.int32),
        pltpu.VMEM((C,), jnp.int32),
        pltpu.VMEM((C,), jnp.int32),
        pltpu.VMEM((C, D), jnp.float32),
        pltpu.VMEM((C, D), jnp.float32),
        pltpu.VMEM_SHARED((NPAD, D), jnp.float32),
        pltpu.SemaphoreType.DMA,
        pltpu.SemaphoreType.DMA,
    ],
)


def _prep_body(feat_ref, deg_ref, xs_ref, ns_ref, nd_ref):
    d_o = deg_ref[0, :N, 0:1]
    d_i = deg_ref[1, :N, 0:1]
    ns = jnp.where(d_o > 0, lax.rsqrt(d_o), 0.0)
    nd = jnp.where(d_i > 0, lax.rsqrt(d_i), 0.0)
    ns_ref[...] = ns
    nd_ref[...] = nd
    xs_ref[...] = feat_ref[...] * ns


_prep_call = pl.pallas_call(
    _prep_body,
    out_shape=(jax.ShapeDtypeStruct((N, D), jnp.float32),
               jax.ShapeDtypeStruct((N, 1), jnp.float32),
               jax.ShapeDtypeStruct((N, 1), jnp.float32)),
)


def _layer1_body(p_ref, nd_ref, ns_ref, w_ref, b_ref, o_ref):
    t = (p_ref[0, :N] + p_ref[1, :N]) * nd_ref[...]
    h = jnp.dot(t, w_ref[...], preferred_element_type=jnp.float32) + b_ref[...]
    o_ref[...] = jnp.maximum(h, 0.0) * ns_ref[...]


_layer1_call = pl.pallas_call(
    _layer1_body,
    out_shape=jax.ShapeDtypeStruct((N, D), jnp.float32),
)


def _layer2_body(p_ref, nd_ref, w_ref, b_ref, o_ref):
    t = (p_ref[0, :N] + p_ref[1, :N]) * nd_ref[...]
    o_ref[...] = jnp.dot(t, w_ref[...], preferred_element_type=jnp.float32) + b_ref[...]


_layer2_call = pl.pallas_call(
    _layer2_body,
    out_shape=jax.ShapeDtypeStruct((N, D), jnp.float32),
)


def kernel(feat, edge_index, W1, b1, W2, b2):
    src = edge_index[0].astype(jnp.int32)
    dst = edge_index[1].astype(jnp.int32)
    zeros_nd = jnp.zeros((NPAD, D), jnp.float32)

    deg = _deg_call(edge_index.reshape(-1).astype(jnp.int32), zeros_nd)
    xs, ns, nd = _prep_call(feat, deg)
    p1 = _agg_call(xs, src, dst, zeros_nd)
    x2 = _layer1_call(p1, nd, ns, W1, b1.reshape(1, D))
    p2 = _agg_call(x2, src, dst, zeros_nd)
    return _layer2_call(p2, nd, W2, b2.reshape(1, D))
